# baseline (speedup 1.0000x reference)
import functools

import jax
import jax.numpy as jnp
from jax import lax
from jax.experimental import pallas as pl
from jax.experimental.pallas import tpu as pltpu
from jax.experimental.pallas import tpu_sc as plsc

_NUM_FACTORS = 32
_BATCH = 16384
_NC = 2
_NS = 16
_NW = _NC * _NS
_BPW = _BATCH // _NW
_L = 16
_NWIN = _BPW // _L


def _sc_body(uid_hbm, iid_hbm, uet_hbm, ub_hbm, iet_hbm, ib_hbm, out_hbm,
             idx_u, idx_q, rows_u, rows_q, bias_u, bias_q, out_v, sem):
    wid = lax.axis_index("s") * _NC + lax.axis_index("c")
    base = wid * _BPW

    cp_u = pltpu.async_copy(uid_hbm.at[pl.ds(base, _BPW)], idx_u, sem)
    cp_q = pltpu.async_copy(iid_hbm.at[pl.ds(base, _BPW)], idx_q, sem)
    cp_u.wait()
    cp_q.wait()

    def step(w, carry):
        sl = pl.ds(w * _L, _L)
        iu = idx_u[sl]
        iq = idx_q[sl]
        copies = [pltpu.async_copy(ub_hbm.at[iu], bias_u, sem),
                  pltpu.async_copy(ib_hbm.at[iq], bias_q, sem)]
        for f in range(_NUM_FACTORS):
            copies.append(pltpu.async_copy(uet_hbm.at[f].at[iu], rows_u.at[f], sem))
            copies.append(pltpu.async_copy(iet_hbm.at[f].at[iq], rows_q.at[f], sem))
        for cp in copies:
            cp.wait()
        acc = bias_u[...] + bias_q[...]
        for f in range(_NUM_FACTORS):
            acc = acc + rows_u[f, :] * rows_q[f, :]
        out_v[sl] = acc
        return carry

    lax.fori_loop(0, _NWIN, step, 0)
    pltpu.sync_copy(out_v, out_hbm.at[pl.ds(base, _BPW)])


_mesh = plsc.VectorSubcoreMesh(core_axis_name="c", subcore_axis_name="s")

_sc_kernel = functools.partial(
    pl.kernel,
    out_type=jax.ShapeDtypeStruct((_BATCH,), jnp.float32),
    mesh=_mesh,
    compiler_params=pltpu.CompilerParams(
        needs_layout_passes=False, use_tc_tiling_on_sc=False),
    scratch_types=[
        pltpu.VMEM((_BPW,), jnp.int32),
        pltpu.VMEM((_BPW,), jnp.int32),
        pltpu.VMEM((_NUM_FACTORS, _L), jnp.float32),
        pltpu.VMEM((_NUM_FACTORS, _L), jnp.float32),
        pltpu.VMEM((_L,), jnp.float32),
        pltpu.VMEM((_L,), jnp.float32),
        pltpu.VMEM((_BPW,), jnp.float32),
        pltpu.SemaphoreType.DMA,
    ],
)(_sc_body)


def kernel(user_id, item_id, user_embeddings, user_bias, item_embeddings, item_bias):
    uid = user_id.astype(jnp.int32)
    iid = item_id.astype(jnp.int32)
    return _sc_kernel(uid, iid, user_embeddings.T, user_bias.reshape(-1),
                      item_embeddings.T, item_bias.reshape(-1))


# speedup vs baseline: 5.7554x; 5.7554x over previous
import functools

import jax
import jax.numpy as jnp
from jax import lax
from jax.experimental import pallas as pl
from jax.experimental.pallas import tpu as pltpu
from jax.experimental.pallas import tpu_sc as plsc

_NUM_FACTORS = 32
_BATCH = 16384
_NC = 2
_NS = 16
_NW = _NC * _NS
_BPW = _BATCH // _NW
_CHUNK = 128
_NCHUNK = _BPW // _CHUNK
_L = 16


def _sc_body(uid_hbm, iid_hbm, ue_hbm, ub_hbm, ie_hbm, ib_hbm, out_hbm,
             idx_u, idx_q, rows_u, rows_q, bias_u, bias_q, out_v, sem):
    wid = lax.axis_index("s") * _NC + lax.axis_index("c")
    base = wid * _BPW

    cp_u = pltpu.async_copy(uid_hbm.at[pl.ds(base, _BPW)], idx_u, sem)
    cp_q = pltpu.async_copy(iid_hbm.at[pl.ds(base, _BPW)], idx_q, sem)
    cp_u.wait()
    cp_q.wait()

    copies = []
    for c in range(_NCHUNK):
        sl = pl.ds(c * _CHUNK, _CHUNK)
        copies.append(pltpu.async_copy(ue_hbm.at[idx_u.at[sl]], rows_u.at[sl], sem))
        copies.append(pltpu.async_copy(ie_hbm.at[idx_q.at[sl]], rows_q.at[sl], sem))
        copies.append(pltpu.async_copy(ub_hbm.at[idx_u.at[sl]], bias_u.at[sl], sem))
        copies.append(pltpu.async_copy(ib_hbm.at[idx_q.at[sl]], bias_q.at[sl], sem))
    for cp in copies:
        cp.wait()

    iota = lax.iota(jnp.int32, _L)
    zero = jnp.zeros((_L,), jnp.int32)

    def step(b, carry):
        sl = pl.ds(b * _L, _L)
        rid = b * _L + iota
        acc = bias_u[sl] + bias_q[sl]
        for f in range(_NUM_FACTORS):
            fv = zero + f
            u = plsc.load_gather(rows_u, [rid, fv])
            q = plsc.load_gather(rows_q, [rid, fv])
            acc = acc + u * q
        out_v[sl] = acc
        return carry

    lax.fori_loop(0, _BPW // _L, step, 0)
    pltpu.sync_copy(out_v, out_hbm.at[pl.ds(base, _BPW)])


_mesh = plsc.VectorSubcoreMesh(core_axis_name="c", subcore_axis_name="s")

_sc_kernel = functools.partial(
    pl.kernel,
    out_type=jax.ShapeDtypeStruct((_BATCH,), jnp.float32),
    mesh=_mesh,
    compiler_params=pltpu.CompilerParams(
        needs_layout_passes=False, use_tc_tiling_on_sc=False),
    scratch_types=[
        pltpu.VMEM((_BPW,), jnp.int32),
        pltpu.VMEM((_BPW,), jnp.int32),
        pltpu.VMEM((_BPW, _NUM_FACTORS), jnp.float32),
        pltpu.VMEM((_BPW, _NUM_FACTORS), jnp.float32),
        pltpu.VMEM((_BPW,), jnp.float32),
        pltpu.VMEM((_BPW,), jnp.float32),
        pltpu.VMEM((_BPW,), jnp.float32),
        pltpu.SemaphoreType.DMA,
    ],
)(_sc_body)


def kernel(user_id, item_id, user_embeddings, user_bias, item_embeddings, item_bias):
    uid = user_id.astype(jnp.int32)
    iid = item_id.astype(jnp.int32)
    return _sc_kernel(uid, iid, user_embeddings, user_bias.reshape(-1),
                      item_embeddings, item_bias.reshape(-1))
